# baseline (speedup 1.0000x reference)
import functools

import jax
import jax.numpy as jnp
from jax import lax
from jax.experimental import pallas as pl
from jax.experimental.pallas import tpu as pltpu
from jax.experimental.pallas import tpu_sc as plsc

B = 1024
D = 64
K = 1000000
BK = 2048
NK = -(-K // BK)
NG = BK // 8
TAIL_G = (K - (NK - 1) * BK) // 8


def _argminmax_body(xt_ref, fqt_ref, posi_ref, negi_ref,
                    maxv, maxg, minv, ming):
    k = pl.program_id(0)

    @pl.when(k == 0)
    def _init():
        maxv[...] = jnp.full((8, B), -jnp.inf, jnp.float32)
        minv[...] = jnp.full((8, B), jnp.inf, jnp.float32)
        maxg[...] = jnp.zeros((8, B), jnp.int32)
        ming[...] = jnp.zeros((8, B), jnp.int32)

    logits = lax.dot_general(
        fqt_ref[...], xt_ref[...],
        dimension_numbers=(((0,), (0,)), ((), ())),
        preferred_element_type=jnp.float32,
    )

    def upd(carry, g):
        mv, gv, nv, hv = carry
        blk = logits[8 * g:8 * g + 8]
        gg = k * NG + g
        up = blk > mv
        mv = jnp.where(up, blk, mv)
        gv = jnp.where(up, gg, gv)
        dn = blk < nv
        nv = jnp.where(dn, blk, nv)
        hv = jnp.where(dn, gg, hv)
        return mv, gv, nv, hv

    carry = (maxv[...], maxg[...], minv[...], ming[...])
    for g in range(TAIL_G):
        carry = upd(carry, g)

    def rest(c):
        for g in range(TAIL_G, NG):
            c = upd(c, g)
        return c

    mv, gv, nv, hv = lax.cond(k < NK - 1, rest, lambda c: c, carry)
    maxv[...], maxg[...] = mv, gv
    minv[...], ming[...] = nv, hv

    @pl.when(k == NK - 1)
    def _fin():
        s_iota = lax.broadcasted_iota(jnp.int32, (8, B), 0)
        big = jnp.int32(2147483647)

        idx = gv * 8 + s_iota
        m = jnp.max(mv, axis=0, keepdims=True)
        cand = jnp.where(mv == m, idx, big)
        posi_ref[...] = jnp.min(cand, axis=0, keepdims=True)

        idxn = hv * 8 + s_iota
        n = jnp.min(nv, axis=0, keepdims=True)
        candn = jnp.where(nv == n, idxn, big)
        negi_ref[...] = jnp.min(candn, axis=0, keepdims=True)


def _argminmax(x, feature_queue):
    return pl.pallas_call(
        _argminmax_body,
        grid=(NK,),
        in_specs=[
            pl.BlockSpec((D, B), lambda k: (0, 0)),
            pl.BlockSpec((D, BK), lambda k: (0, k)),
        ],
        out_specs=[
            pl.BlockSpec((1, B), lambda k: (0, 0)),
            pl.BlockSpec((1, B), lambda k: (0, 0)),
        ],
        out_shape=[
            jax.ShapeDtypeStruct((1, B), jnp.int32),
            jax.ShapeDtypeStruct((1, B), jnp.int32),
        ],
        scratch_shapes=[
            pltpu.VMEM((8, B), jnp.float32),
            pltpu.VMEM((8, B), jnp.int32),
            pltpu.VMEM((8, B), jnp.float32),
            pltpu.VMEM((8, B), jnp.int32),
        ],
        compiler_params=pltpu.CompilerParams(
            dimension_semantics=("arbitrary",),
        ),
    )(x.T, feature_queue.T)


def _make_sc_gather(n_idx):
    info = plsc.get_sparse_core_info()
    nw = info.num_cores * info.num_subcores
    b_per_w = n_idx // nw
    mesh = plsc.VectorSubcoreMesh(core_axis_name="c", subcore_axis_name="s")

    @functools.partial(
        pl.kernel,
        out_type=jax.ShapeDtypeStruct((n_idx, D), jnp.float32),
        mesh=mesh,
        scratch_types=[
            pltpu.VMEM((b_per_w,), jnp.int32),
            pltpu.VMEM((b_per_w, D), jnp.float32),
            pltpu.SemaphoreType.DMA,
        ],
        compiler_params=pltpu.CompilerParams(use_tc_tiling_on_sc=False),
    )
    def gather(table_hbm, idx_hbm, out_hbm, idx_v, rows_v, sem):
        wid = lax.axis_index("s") * info.num_cores + lax.axis_index("c")
        base = wid * b_per_w
        pltpu.sync_copy(idx_hbm.at[pl.ds(base, b_per_w)], idx_v)
        pltpu.async_copy(table_hbm.at[idx_v], rows_v, sem).wait()
        pltpu.sync_copy(rows_v, out_hbm.at[pl.ds(base, b_per_w)])

    return gather


def kernel(x, feature_queue, rep_queue):
    pos_idx, neg_idx = _argminmax(x, feature_queue)
    idx = jnp.concatenate([pos_idx.reshape(B), neg_idx.reshape(B)])
    reps = _make_sc_gather(2 * B)(rep_queue, idx)
    return reps[:B], reps[B:]


# speedup vs baseline: 1.2939x; 1.2939x over previous
import functools

import jax
import jax.numpy as jnp
from jax import lax
from jax.experimental import pallas as pl
from jax.experimental.pallas import tpu as pltpu
from jax.experimental.pallas import tpu_sc as plsc

B = 1024
D = 64
K = 1000000
BK = 1024
NK = -(-K // BK)
NG = BK // 8
TAIL_G = (K - (NK - 1) * BK) // 8


def _argminmax_body(xt_ref, fqt_ref, rqt_ref, posi_ref, negi_ref, rq128_ref,
                    maxv, maxg, minv, ming):
    k = pl.program_id(0)

    @pl.when(k == 0)
    def _init():
        maxv[...] = jnp.full((8, B), -jnp.inf, jnp.float32)
        minv[...] = jnp.full((8, B), jnp.inf, jnp.float32)
        maxg[...] = jnp.zeros((8, B), jnp.int32)
        ming[...] = jnp.zeros((8, B), jnp.int32)

    rq128_ref[:, 0:D] = jnp.transpose(rqt_ref[...])

    logits = lax.dot_general(
        fqt_ref[...], xt_ref[...],
        dimension_numbers=(((0,), (0,)), ((), ())),
        preferred_element_type=jnp.float32,
    )

    def upd(carry, g):
        mv, gv, nv, hv = carry
        blk = logits[8 * g:8 * g + 8]
        gg = k * NG + g
        up = blk > mv
        mv = jnp.where(up, blk, mv)
        gv = jnp.where(up, gg, gv)
        dn = blk < nv
        nv = jnp.where(dn, blk, nv)
        hv = jnp.where(dn, gg, hv)
        return mv, gv, nv, hv

    carry = (maxv[...], maxg[...], minv[...], ming[...])
    for g in range(TAIL_G):
        carry = upd(carry, g)

    def rest(c):
        for g in range(TAIL_G, NG):
            c = upd(c, g)
        return c

    mv, gv, nv, hv = lax.cond(k < NK - 1, rest, lambda c: c, carry)
    maxv[...], maxg[...] = mv, gv
    minv[...], ming[...] = nv, hv

    @pl.when(k == NK - 1)
    def _fin():
        s_iota = lax.broadcasted_iota(jnp.int32, (8, B), 0)
        big = jnp.int32(2147483647)

        idx = gv * 8 + s_iota
        m = jnp.max(mv, axis=0, keepdims=True)
        cand = jnp.where(mv == m, idx, big)
        posi_ref[...] = jnp.min(cand, axis=0, keepdims=True)

        idxn = hv * 8 + s_iota
        n = jnp.min(nv, axis=0, keepdims=True)
        candn = jnp.where(nv == n, idxn, big)
        negi_ref[...] = jnp.min(candn, axis=0, keepdims=True)


def _argminmax(x, feature_queue, rep_queue):
    return pl.pallas_call(
        _argminmax_body,
        grid=(NK,),
        in_specs=[
            pl.BlockSpec((D, B), lambda k: (0, 0)),
            pl.BlockSpec((D, BK), lambda k: (0, k)),
            pl.BlockSpec((D, BK), lambda k: (0, k)),
        ],
        out_specs=[
            pl.BlockSpec((1, B), lambda k: (0, 0)),
            pl.BlockSpec((1, B), lambda k: (0, 0)),
            pl.BlockSpec((BK, 2 * D), lambda k: (k, 0)),
        ],
        out_shape=[
            jax.ShapeDtypeStruct((1, B), jnp.int32),
            jax.ShapeDtypeStruct((1, B), jnp.int32),
            jax.ShapeDtypeStruct((K, 2 * D), jnp.float32),
        ],
        scratch_shapes=[
            pltpu.VMEM((8, B), jnp.float32),
            pltpu.VMEM((8, B), jnp.int32),
            pltpu.VMEM((8, B), jnp.float32),
            pltpu.VMEM((8, B), jnp.int32),
        ],
        compiler_params=pltpu.CompilerParams(
            dimension_semantics=("arbitrary",),
        ),
    )(x.T, feature_queue.T, rep_queue.T)


def _make_sc_gather(n_idx, width):
    info = plsc.get_sparse_core_info()
    nw = info.num_cores * info.num_subcores
    b_per_w = n_idx // nw
    mesh = plsc.VectorSubcoreMesh(core_axis_name="c", subcore_axis_name="s")

    @functools.partial(
        pl.kernel,
        out_type=jax.ShapeDtypeStruct((n_idx, width), jnp.float32),
        mesh=mesh,
        scratch_types=[
            pltpu.VMEM((b_per_w,), jnp.int32),
            pltpu.VMEM((b_per_w, width), jnp.float32),
            pltpu.SemaphoreType.DMA,
        ],
    )
    def gather(table_hbm, idx_hbm, out_hbm, idx_v, rows_v, sem):
        wid = lax.axis_index("s") * info.num_cores + lax.axis_index("c")
        base = wid * b_per_w
        pltpu.sync_copy(idx_hbm.at[pl.ds(base, b_per_w)], idx_v)
        pltpu.async_copy(table_hbm.at[idx_v], rows_v, sem).wait()
        pltpu.sync_copy(rows_v, out_hbm.at[pl.ds(base, b_per_w)])

    return gather


def kernel(x, feature_queue, rep_queue):
    pos_idx, neg_idx, rq128 = _argminmax(x, feature_queue, rep_queue)
    idx = jnp.concatenate([pos_idx.reshape(B), neg_idx.reshape(B)])
    reps = _make_sc_gather(2 * B, 2 * D)(rq128, idx)[:, :D]
    return reps[:B], reps[B:]
